# baseline (speedup 1.0000x reference)
import functools

import jax
import jax.numpy as jnp
from jax import lax
from jax.experimental import pallas as pl
from jax.experimental.pallas import tpu as pltpu
from jax.experimental.pallas import tpu_sc as plsc

_K = 128
_G = 10
_KW = 64


@functools.lru_cache(maxsize=None)
def _make_sc_gather(n_items, n_users, B, E, d, n_negs):
    info = plsc.get_sparse_core_info()
    NC, NS = info.num_cores, info.num_subcores
    NW = NC * NS
    EW = E // NW
    NCH = EW // _K
    BW = B // NW
    NQ = BW // _K
    NEG = B * n_negs
    NEGW = NEG // NW
    NNCH = NEGW // _K
    RPT = B // NS

    mesh = plsc.VectorSubcoreMesh(core_axis_name="c", subcore_axis_name="s")

    @functools.partial(
        pl.kernel,
        out_type=(
            jax.ShapeDtypeStruct((NC, B, 128), jnp.float32),
            jax.ShapeDtypeStruct((B, 128), jnp.float32),
            jax.ShapeDtypeStruct((B, 128), jnp.float32),
            jax.ShapeDtypeStruct((NEG, 128), jnp.float32),
            jax.ShapeDtypeStruct((B,), jnp.float32),
            jax.ShapeDtypeStruct((B,), jnp.float32),
            jax.ShapeDtypeStruct((NEG,), jnp.float32),
        ),
        mesh=mesh,
        compiler_params=pltpu.CompilerParams(use_tc_tiling_on_sc=False),
        scratch_types=[
            pltpu.VMEM((_G * _K,), jnp.int32),
            pltpu.VMEM((_G, _K), jnp.int32),
            pltpu.VMEM((4, _K, d), jnp.float32),
            pltpu.VMEM((2, _KW, 128), jnp.int32).update(dtype=jnp.float32)
            if False else pltpu.VMEM((2, _KW, 128), jnp.float32),
            pltpu.VMEM(((2 * (B // _K // NW) + (NEG // _K // NW)) * _K,),
                       jnp.int32),
            pltpu.VMEM((_K,), jnp.float32),
            pltpu.VMEM((_K,), jnp.float32),
            pltpu.VMEM_SHARED((B, d), jnp.float32),
            pltpu.SemaphoreType.DMA,
            pltpu.SemaphoreType.DMA,
            pltpu.SemaphoreType.DMA,
            pltpu.SemaphoreType.DMA,
            pltpu.SemaphoreType.DMA,
            pltpu.SemaphoreType.DMA,
            pltpu.SemaphoreType.DMA,
            pltpu.SemaphoreType.DMA,
            pltpu.SemaphoreType.DMA,
            pltpu.SemaphoreType.DMA,
            pltpu.SemaphoreType.DMA,
            pltpu.SemaphoreType.DMA,
            pltpu.SemaphoreType.DMA,
            pltpu.SemaphoreType.DMA,
        ],
    )
    def sc_kernel(p_hbm, q_hbm, bu_hbm, bi_hbm, iu1, us1, i1, u1, ineg1, z_hbm,
                  psum_out, q_out, pself_out, qneg_out, bu_out, bi_out, bineg_out,
                  iu_g, us_g, ring, wide, idx_v, bva, bvb, psum_sh,
                  g0, g1, g2, g3, s0, s1, s2, s3,
                  rsem_a, rsem_b, wsem_a, wsem_b, bsem_a, bsem_b):
        gsems = (g0, g1, g2, g3)
        ssems = (s0, s1, s2, s3)
        gbufs = tuple(ring.at[i] for i in range(4))
        wbufs = (wide.at[0], wide.at[1])
        rsems = (rsem_a, rsem_b)
        wsems = (wsem_a, wsem_b)
        bsems = (bsem_a, bsem_b)
        bbufs = (bva, bvb)
        cid = lax.axis_index("c")
        sid = lax.axis_index("s")
        wid = cid * NS + sid

        pltpu.sync_copy(z_hbm, gbufs[0])
        for j in range(RPT // _K):
            pltpu.sync_copy(gbufs[0], psum_sh.at[pl.ds(sid * RPT + j * _K, _K)])

        pltpu.sync_copy(i1.at[pl.ds(wid * BW, BW)], idx_v.at[pl.ds(0, BW)])
        pltpu.sync_copy(u1.at[pl.ds(wid * BW, BW)], idx_v.at[pl.ds(BW, BW)])
        pltpu.sync_copy(ineg1.at[pl.ds(wid * NEGW, NEGW)],
                        idx_v.at[pl.ds(2 * BW, NEGW)])

        tasks = []
        for j in range(BW // _KW):
            base = wid * BW + j * _KW
            tasks.append((True, q_hbm, j * _KW, _KW,
                          q_out.at[pl.ds(base, _KW)],
                          bi_hbm, j * _KW, bi_out.at[pl.ds(base, _KW)]))
        for j in range(NQ):
            base = wid * BW + j * _K
            tasks.append((False, p_hbm, j * _K, _K,
                          pself_out.at[pl.ds(base, _K), pl.ds(0, d)],
                          bu_hbm, BW + j * _K, bu_out.at[pl.ds(base, _K)]))
        for j in range(NEGW // _KW):
            base = wid * NEGW + j * _KW
            tasks.append((True, q_hbm, 2 * BW + j * _KW, _KW,
                          qneg_out.at[pl.ds(base, _KW)],
                          bi_hbm, 2 * BW + j * _KW,
                          bineg_out.at[pl.ds(base, _KW)]))

        wcnt = [0]
        ncnt = [0]
        bcnt = [0]

        def fire(t):
            w, tab, ioff, nrows, _, btab, bioff, _ = tasks[t]
            if w:
                buf = wbufs[wcnt[0] % 2]
                sem = wsems[wcnt[0] % 2]
                wcnt[0] += 1
                src = tab.at[idx_v.at[pl.ds(ioff, nrows)]]
                dsc = [pltpu.async_copy(src.at[pl.ds(0, nrows)]
                                        if False else src, buf, sem)]
            else:
                buf = gbufs[ncnt[0] % 2]
                sem = rsems[ncnt[0] % 2]
                ncnt[0] += 1
                dsc = [pltpu.async_copy(tab.at[idx_v.at[pl.ds(ioff, nrows)]],
                                        buf, sem)]
            bb = bbufs[bcnt[0] % 2]
            bs = bsems[bcnt[0] % 2]
            bcnt[0] += 1
            dsc.append(pltpu.async_copy(
                bi_hbm.at[idx_v.at[pl.ds(bioff, nrows)]]
                if btab is bi_hbm else btab.at[idx_v.at[pl.ds(bioff, nrows)]],
                bb.at[pl.ds(0, nrows)], bs))
            return dsc, (buf, bb.at[pl.ds(0, nrows)])

        pend = fire(0)
        wso = [0]
        nso = [0]
        bso = [0]

        for t in range(len(tasks)):
            nxt = fire(t + 1) if t + 1 < len(tasks) else None
            descs, (buf, bb) = pend
            for dsc in descs:
                dsc.wait()
            pend = nxt
            _, _, _, nrows, out, btab, _, bout = tasks[t]
            pltpu.sync_copy(buf, out)
            pltpu.sync_copy(bb, bout)

        plsc.subcore_barrier()

        @pl.loop(0, NCH, step=_G)
        def _edges(j):
            ebase = wid * EW + j * _K
            pltpu.sync_copy(iu1.at[pl.ds(ebase, _G * _K)], iu_g)
            sdescs = [pltpu.async_copy(us1.at[pl.ds(ebase + k * _K, _K)],
                                       us_g.at[k], rsem_a) for k in range(_G)]
            for dsc in sdescs:
                dsc.wait()
            _A = 3
            for k0 in range(_A):
                pltpu.async_copy(p_hbm.at[iu_g.at[pl.ds(k0 * _K, _K)]],
                                 gbufs[k0 % 4], gsems[k0 % 4])
            sd = {}
            for k in range(_G):
                cur = gbufs[k % 4]
                pltpu.make_async_copy(p_hbm.at[iu_g.at[pl.ds(k * _K, _K)]],
                                      cur, gsems[k % 4]).wait()
                sd[k] = pltpu.async_copy(cur, psum_sh.at[us_g.at[k]],
                                         ssems[k % 4], add=True)
                if k + _A < _G:
                    kn = k + _A
                    if kn - 4 >= 0:
                        sd.pop(kn - 4).wait()
                    pltpu.async_copy(p_hbm.at[iu_g.at[pl.ds(kn * _K, _K)]],
                                     gbufs[kn % 4], gsems[kn % 4])
            for dsc in sd.values():
                dsc.wait()

        plsc.subcore_barrier()

        pltpu.sync_copy(psum_sh.at[pl.ds(sid * RPT, RPT)],
                        psum_out.at[cid, pl.ds(sid * RPT, RPT), pl.ds(0, d)])

    return sc_kernel, NC, NW, NCH, NNCH, NQ


def _tc_score(psum, q, pself, qneg3, bu, bi, bineg2, B, d, n_negs, NC):
    BLK = 2048
    grid = (B // BLK,)

    def body(psum_ref, q_ref, pself_ref, qneg_ref, bu_ref, bi_ref, bineg_ref,
             r_ref, rneg_ref):
        p_sum = psum_ref[0, :, :d]
        for c in range(1, NC):
            p_sum = p_sum + psum_ref[c, :, :d]
        p_ctx = p_sum - pself_ref[:, :d]
        pq = jnp.sum(p_ctx * q_ref[:, :d], axis=1)
        r_ref[...] = bu_ref[...] + bi_ref[...] + pq
        pqn = jnp.sum(p_ctx[:, None, :] * qneg_ref[:, :, :d], axis=2)
        rneg_ref[...] = bu_ref[...][:, None] + bineg_ref[...] + pqn

    return pl.pallas_call(
        body,
        grid=grid,
        in_specs=[
            pl.BlockSpec((NC, BLK, 128), lambda i: (0, i, 0)),
            pl.BlockSpec((BLK, 128), lambda i: (i, 0)),
            pl.BlockSpec((BLK, 128), lambda i: (i, 0)),
            pl.BlockSpec((BLK, n_negs, 128), lambda i: (i, 0, 0)),
            pl.BlockSpec((BLK,), lambda i: (i,)),
            pl.BlockSpec((BLK,), lambda i: (i,)),
            pl.BlockSpec((BLK, n_negs), lambda i: (i, 0)),
        ],
        out_specs=[
            pl.BlockSpec((BLK,), lambda i: (i,)),
            pl.BlockSpec((BLK, n_negs), lambda i: (i, 0)),
        ],
        out_shape=[
            jax.ShapeDtypeStruct((B,), jnp.float32),
            jax.ShapeDtypeStruct((B, n_negs), jnp.float32),
        ],
    )(psum, q, pself, qneg3, bu, bi, bineg2)


def kernel(P_table, Q_table, b_u, b_i, I, U, I_neg, I_U, U_idx):
    B = I.shape[0]
    n_negs = I_neg.shape[1]
    E = I_U.shape[0]
    n_items, d = P_table.shape
    n_users = b_u.shape[0]

    sc_kernel, NC, NW, NCH, NNCH, NQ = _make_sc_gather(
        n_items, n_users, B, E, d, n_negs)

    iu1 = I_U.astype(jnp.int32)
    us1 = U_idx.astype(jnp.int32)
    i1 = I.astype(jnp.int32)
    u1 = U.astype(jnp.int32)
    ineg1 = I_neg.astype(jnp.int32).reshape(-1)
    zeros = jnp.zeros((_K, d), jnp.float32)

    q_pad = jnp.pad(Q_table, ((0, 0), (0, 128 - d)))
    psum, q, pself, qneg, bu, bi, bineg = sc_kernel(
        P_table, q_pad, b_u, b_i, iu1, us1, i1, u1, ineg1, zeros)

    qneg3 = qneg.reshape(B, n_negs, 128)
    bineg2 = bineg.reshape(B, n_negs)
    r, rneg = _tc_score(psum, q, pself, qneg3, bu, bi, bineg2, B, d, n_negs, NC)
    return (r, rneg)


# speedup vs baseline: 1.0313x; 1.0313x over previous
import functools

import jax
import jax.numpy as jnp
from jax import lax
from jax.experimental import pallas as pl
from jax.experimental.pallas import tpu as pltpu
from jax.experimental.pallas import tpu_sc as plsc

_K = 128
_G = 10
_KW = 64


@functools.lru_cache(maxsize=None)
def _make_sc_gather(n_items, n_users, B, E, d, n_negs):
    info = plsc.get_sparse_core_info()
    NC, NS = info.num_cores, info.num_subcores
    NW = NC * NS
    EW = E // NW
    NCH = EW // _K
    BW = B // NW
    NQ = BW // _K
    NEG = B * n_negs
    NEGW = NEG // NW
    NNCH = NEGW // _K
    RPT = B // NS

    mesh = plsc.VectorSubcoreMesh(core_axis_name="c", subcore_axis_name="s")

    @functools.partial(
        pl.kernel,
        out_type=(
            jax.ShapeDtypeStruct((NC, B, 128), jnp.float32),
            jax.ShapeDtypeStruct((B, 128), jnp.float32),
            jax.ShapeDtypeStruct((B, 128), jnp.float32),
            jax.ShapeDtypeStruct((NEG, 128), jnp.float32),
            jax.ShapeDtypeStruct((B,), jnp.float32),
            jax.ShapeDtypeStruct((B,), jnp.float32),
            jax.ShapeDtypeStruct((NEG,), jnp.float32),
        ),
        mesh=mesh,
        compiler_params=pltpu.CompilerParams(use_tc_tiling_on_sc=False),
        scratch_types=[
            pltpu.VMEM((_G * _K,), jnp.int32),
            pltpu.VMEM((_G, _K), jnp.int32),
            pltpu.VMEM((4, _K, d), jnp.float32),
            pltpu.VMEM(((2 * (B // _K // NW) + (NEG // _K // NW)) * _K,),
                       jnp.int32),
            pltpu.VMEM((_K,), jnp.float32),
            pltpu.VMEM((_K,), jnp.float32),
            pltpu.VMEM_SHARED((B, d), jnp.float32),
            pltpu.SemaphoreType.DMA,
            pltpu.SemaphoreType.DMA,
            pltpu.SemaphoreType.DMA,
            pltpu.SemaphoreType.DMA,
            pltpu.SemaphoreType.DMA,
            pltpu.SemaphoreType.DMA,
            pltpu.SemaphoreType.DMA,
            pltpu.SemaphoreType.DMA,
            pltpu.SemaphoreType.DMA,
            pltpu.SemaphoreType.DMA,
            pltpu.SemaphoreType.DMA,
            pltpu.SemaphoreType.DMA,
        ],
    )
    def sc_kernel(p_hbm, q_hbm, bu_hbm, bi_hbm, iu1, us1, i1, u1, ineg1, z_hbm,
                  psum_out, q_out, pself_out, qneg_out, bu_out, bi_out, bineg_out,
                  iu_g, us_g, ring, idx_v, bva, bvb, psum_sh,
                  g0, g1, g2, g3, s0, s1, s2, s3,
                  rsem_a, rsem_b, bsem_a, bsem_b):
        gsems = (g0, g1, g2, g3)
        ssems = (s0, s1, s2, s3)
        gbufs = tuple(ring.at[i] for i in range(4))
        rsems = (rsem_a, rsem_b)
        bsems = (bsem_a, bsem_b)
        bbufs = (bva, bvb)
        cid = lax.axis_index("c")
        sid = lax.axis_index("s")
        wid = cid * NS + sid

        pltpu.sync_copy(z_hbm, gbufs[0])
        for j in range(RPT // _K):
            pltpu.sync_copy(gbufs[0], psum_sh.at[pl.ds(sid * RPT + j * _K, _K)])

        pltpu.sync_copy(i1.at[pl.ds(wid * BW, BW)], idx_v.at[pl.ds(0, BW)])
        pltpu.sync_copy(u1.at[pl.ds(wid * BW, BW)], idx_v.at[pl.ds(BW, BW)])
        pltpu.sync_copy(ineg1.at[pl.ds(wid * NEGW, NEGW)],
                        idx_v.at[pl.ds(2 * BW, NEGW)])

        tasks = []
        for j in range(NQ):
            base = wid * BW + j * _K
            tasks.append((q_hbm, j * _K,
                          q_out.at[pl.ds(base, _K), pl.ds(0, d)],
                          bi_hbm, j * _K, bi_out.at[pl.ds(base, _K)]))
            tasks.append((p_hbm, j * _K,
                          pself_out.at[pl.ds(base, _K), pl.ds(0, d)],
                          bu_hbm, BW + j * _K, bu_out.at[pl.ds(base, _K)]))
        for j in range(NNCH):
            base = wid * NEGW + j * _K
            tasks.append((q_hbm, 2 * BW + j * _K,
                          qneg_out.at[pl.ds(base, _K), pl.ds(0, d)],
                          bi_hbm, 2 * BW + j * _K,
                          bineg_out.at[pl.ds(base, _K)]))

        def fire(t):
            tab, ioff, _, btab, bioff, _ = tasks[t]
            return [
                pltpu.async_copy(tab.at[idx_v.at[pl.ds(ioff, _K)]],
                                 gbufs[t % 2], rsems[t % 2]),
                pltpu.async_copy(btab.at[idx_v.at[pl.ds(bioff, _K)]],
                                 bbufs[t % 2], bsems[t % 2]),
            ]

        pend = fire(0)
        for t in range(len(tasks)):
            nxt = fire(t + 1) if t + 1 < len(tasks) else None
            for dsc in pend:
                dsc.wait()
            pend = nxt
            _, _, out, _, _, bout = tasks[t]
            pltpu.sync_copy(gbufs[t % 2], out)
            pltpu.sync_copy(bbufs[t % 2], bout)

        plsc.subcore_barrier()

        @pl.loop(0, NCH, step=_G)
        def _edges(j):
            ebase = wid * EW + j * _K
            pltpu.sync_copy(iu1.at[pl.ds(ebase, _G * _K)], iu_g)
            sdescs = [pltpu.async_copy(us1.at[pl.ds(ebase + k * _K, _K)],
                                       us_g.at[k], rsem_a) for k in range(_G)]
            for dsc in sdescs:
                dsc.wait()
            _A = 3
            for k0 in range(_A):
                pltpu.async_copy(p_hbm.at[iu_g.at[pl.ds(k0 * _K, _K)]],
                                 gbufs[k0 % 4], gsems[k0 % 4])
            sd = {}
            for k in range(_G):
                cur = gbufs[k % 4]
                pltpu.make_async_copy(p_hbm.at[iu_g.at[pl.ds(k * _K, _K)]],
                                      cur, gsems[k % 4]).wait()
                sd[k] = pltpu.async_copy(cur, psum_sh.at[us_g.at[k]],
                                         ssems[k % 4], add=True)
                if k + _A < _G:
                    kn = k + _A
                    if kn - 4 >= 0:
                        sd.pop(kn - 4).wait()
                    pltpu.async_copy(p_hbm.at[iu_g.at[pl.ds(kn * _K, _K)]],
                                     gbufs[kn % 4], gsems[kn % 4])
            for dsc in sd.values():
                dsc.wait()

        plsc.subcore_barrier()

        pltpu.sync_copy(psum_sh.at[pl.ds(sid * RPT, RPT)],
                        psum_out.at[cid, pl.ds(sid * RPT, RPT), pl.ds(0, d)])

    return sc_kernel, NC, NW, NCH, NNCH, NQ


def _tc_score(psum, q, pself, qneg3, bu, bi, bineg2, B, d, n_negs, NC):
    BLK = 2048
    grid = (B // BLK,)

    def body(psum_ref, q_ref, pself_ref, qneg_ref, bu_ref, bi_ref, bineg_ref,
             r_ref, rneg_ref):
        p_sum = psum_ref[0, :, :d]
        for c in range(1, NC):
            p_sum = p_sum + psum_ref[c, :, :d]
        p_ctx = p_sum - pself_ref[:, :d]
        pq = jnp.sum(p_ctx * q_ref[:, :d], axis=1)
        r_ref[...] = bu_ref[...] + bi_ref[...] + pq
        pqn = jnp.sum(p_ctx[:, None, :] * qneg_ref[:, :, :d], axis=2)
        rneg_ref[...] = bu_ref[...][:, None] + bineg_ref[...] + pqn

    return pl.pallas_call(
        body,
        grid=grid,
        in_specs=[
            pl.BlockSpec((NC, BLK, 128), lambda i: (0, i, 0)),
            pl.BlockSpec((BLK, 128), lambda i: (i, 0)),
            pl.BlockSpec((BLK, 128), lambda i: (i, 0)),
            pl.BlockSpec((BLK, n_negs, 128), lambda i: (i, 0, 0)),
            pl.BlockSpec((BLK,), lambda i: (i,)),
            pl.BlockSpec((BLK,), lambda i: (i,)),
            pl.BlockSpec((BLK, n_negs), lambda i: (i, 0)),
        ],
        out_specs=[
            pl.BlockSpec((BLK,), lambda i: (i,)),
            pl.BlockSpec((BLK, n_negs), lambda i: (i, 0)),
        ],
        out_shape=[
            jax.ShapeDtypeStruct((B,), jnp.float32),
            jax.ShapeDtypeStruct((B, n_negs), jnp.float32),
        ],
    )(psum, q, pself, qneg3, bu, bi, bineg2)


def kernel(P_table, Q_table, b_u, b_i, I, U, I_neg, I_U, U_idx):
    B = I.shape[0]
    n_negs = I_neg.shape[1]
    E = I_U.shape[0]
    n_items, d = P_table.shape
    n_users = b_u.shape[0]

    sc_kernel, NC, NW, NCH, NNCH, NQ = _make_sc_gather(
        n_items, n_users, B, E, d, n_negs)

    iu1 = I_U.astype(jnp.int32)
    us1 = U_idx.astype(jnp.int32)
    i1 = I.astype(jnp.int32)
    u1 = U.astype(jnp.int32)
    ineg1 = I_neg.astype(jnp.int32).reshape(-1)
    zeros = jnp.zeros((_K, d), jnp.float32)

    psum, q, pself, qneg, bu, bi, bineg = sc_kernel(
        P_table, Q_table, b_u, b_i, iu1, us1, i1, u1, ineg1, zeros)

    qneg3 = qneg.reshape(B, n_negs, 128)
    bineg2 = bineg.reshape(B, n_negs)
    r, rneg = _tc_score(psum, q, pself, qneg3, bu, bi, bineg2, B, d, n_negs, NC)
    return (r, rneg)


# speedup vs baseline: 1.0546x; 1.0226x over previous
import functools

import jax
import jax.numpy as jnp
from jax import lax
from jax.experimental import pallas as pl
from jax.experimental.pallas import tpu as pltpu
from jax.experimental.pallas import tpu_sc as plsc

_K = 128
_G = 10
_KW = 64


@functools.lru_cache(maxsize=None)
def _make_sc_gather(n_items, n_users, B, E, d, n_negs):
    info = plsc.get_sparse_core_info()
    NC, NS = info.num_cores, info.num_subcores
    NW = NC * NS
    EW = E // NW
    NCH = EW // _K
    BW = B // NW
    NQ = BW // _K
    NEG = B * n_negs
    NEGW = NEG // NW
    NNCH = NEGW // _K
    RPT = B // NS

    mesh = plsc.VectorSubcoreMesh(core_axis_name="c", subcore_axis_name="s")

    @functools.partial(
        pl.kernel,
        out_type=(
            jax.ShapeDtypeStruct((NC, B, 128), jnp.float32),
            jax.ShapeDtypeStruct((B, 128), jnp.float32),
            jax.ShapeDtypeStruct((B, 128), jnp.float32),
            jax.ShapeDtypeStruct((NEG, 128), jnp.float32),
            jax.ShapeDtypeStruct((B,), jnp.float32),
            jax.ShapeDtypeStruct((B,), jnp.float32),
            jax.ShapeDtypeStruct((NEG,), jnp.float32),
        ),
        mesh=mesh,
        compiler_params=pltpu.CompilerParams(use_tc_tiling_on_sc=False),
        scratch_types=[
            pltpu.VMEM((_G * _K,), jnp.int32),
            pltpu.VMEM((_G, _K), jnp.int32),
            pltpu.VMEM((5, _K, d), jnp.float32),
            pltpu.VMEM(((2 * (B // _K // NW) + (NEG // _K // NW)) * _K,),
                       jnp.int32),
            pltpu.VMEM((_K,), jnp.float32),
            pltpu.VMEM((_K,), jnp.float32),
            pltpu.VMEM_SHARED((B, d), jnp.float32),
            pltpu.SemaphoreType.DMA,
            pltpu.SemaphoreType.DMA,
            pltpu.SemaphoreType.DMA,
            pltpu.SemaphoreType.DMA,
            pltpu.SemaphoreType.DMA,
            pltpu.SemaphoreType.DMA,
            pltpu.SemaphoreType.DMA,
            pltpu.SemaphoreType.DMA,
            pltpu.SemaphoreType.DMA,
            pltpu.SemaphoreType.DMA,
            pltpu.SemaphoreType.DMA,
            pltpu.SemaphoreType.DMA,
            pltpu.SemaphoreType.DMA,
            pltpu.SemaphoreType.DMA,
        ],
    )
    def sc_kernel(p_hbm, q_hbm, bu_hbm, bi_hbm, iu1, us1, i1, u1, ineg1, z_hbm,
                  psum_out, q_out, pself_out, qneg_out, bu_out, bi_out, bineg_out,
                  iu_g, us_g, ring, idx_v, bva, bvb, psum_sh,
                  g0, g1, g2, g3, g4, s0, s1, s2, s3, s4,
                  rsem_a, rsem_b, bsem_a, bsem_b):
        gsems = (g0, g1, g2, g3, g4)
        ssems = (s0, s1, s2, s3, s4)
        gbufs = tuple(ring.at[i] for i in range(5))
        rsems = (rsem_a, rsem_b)
        bsems = (bsem_a, bsem_b)
        bbufs = (bva, bvb)
        cid = lax.axis_index("c")
        sid = lax.axis_index("s")
        wid = cid * NS + sid

        pltpu.sync_copy(z_hbm, gbufs[0])
        for j in range(RPT // _K):
            pltpu.sync_copy(gbufs[0], psum_sh.at[pl.ds(sid * RPT + j * _K, _K)])

        pltpu.sync_copy(i1.at[pl.ds(wid * BW, BW)], idx_v.at[pl.ds(0, BW)])
        pltpu.sync_copy(u1.at[pl.ds(wid * BW, BW)], idx_v.at[pl.ds(BW, BW)])
        pltpu.sync_copy(ineg1.at[pl.ds(wid * NEGW, NEGW)],
                        idx_v.at[pl.ds(2 * BW, NEGW)])

        tasks = []
        for j in range(NQ):
            base = wid * BW + j * _K
            tasks.append((q_hbm, j * _K,
                          q_out.at[pl.ds(base, _K), pl.ds(0, d)],
                          bi_hbm, j * _K, bi_out.at[pl.ds(base, _K)]))
            tasks.append((p_hbm, j * _K,
                          pself_out.at[pl.ds(base, _K), pl.ds(0, d)],
                          bu_hbm, BW + j * _K, bu_out.at[pl.ds(base, _K)]))
        for j in range(NNCH):
            base = wid * NEGW + j * _K
            tasks.append((q_hbm, 2 * BW + j * _K,
                          qneg_out.at[pl.ds(base, _K), pl.ds(0, d)],
                          bi_hbm, 2 * BW + j * _K,
                          bineg_out.at[pl.ds(base, _K)]))

        def fire(t):
            tab, ioff, _, btab, bioff, _ = tasks[t]
            return [
                pltpu.async_copy(tab.at[idx_v.at[pl.ds(ioff, _K)]],
                                 gbufs[t % 2], rsems[t % 2]),
                pltpu.async_copy(btab.at[idx_v.at[pl.ds(bioff, _K)]],
                                 bbufs[t % 2], bsems[t % 2]),
            ]

        pend = fire(0)
        for t in range(len(tasks)):
            nxt = fire(t + 1) if t + 1 < len(tasks) else None
            for dsc in pend:
                dsc.wait()
            pend = nxt
            _, _, out, _, _, bout = tasks[t]
            pltpu.sync_copy(gbufs[t % 2], out)
            pltpu.sync_copy(bbufs[t % 2], bout)

        plsc.subcore_barrier()

        @pl.loop(0, NCH, step=_G)
        def _edges(j):
            ebase = wid * EW + j * _K
            pltpu.sync_copy(iu1.at[pl.ds(ebase, _G * _K)], iu_g)
            sdescs = [pltpu.async_copy(us1.at[pl.ds(ebase + k * _K, _K)],
                                       us_g.at[k], rsem_a) for k in range(_G)]
            for dsc in sdescs:
                dsc.wait()
            _A = 3
            for k0 in range(_A):
                pltpu.async_copy(p_hbm.at[iu_g.at[pl.ds(k0 * _K, _K)]],
                                 gbufs[k0 % 5], gsems[k0 % 5])
            sd = {}
            for k in range(_G):
                cur = gbufs[k % 5]
                pltpu.make_async_copy(p_hbm.at[iu_g.at[pl.ds(k * _K, _K)]],
                                      cur, gsems[k % 5]).wait()
                sd[k] = pltpu.async_copy(cur, psum_sh.at[us_g.at[k]],
                                         ssems[k % 5], add=True)
                if k + _A < _G:
                    kn = k + _A
                    if kn - 5 >= 0:
                        sd.pop(kn - 5).wait()
                    pltpu.async_copy(p_hbm.at[iu_g.at[pl.ds(kn * _K, _K)]],
                                     gbufs[kn % 5], gsems[kn % 5])
            for dsc in sd.values():
                dsc.wait()

        plsc.subcore_barrier()

        pltpu.sync_copy(psum_sh.at[pl.ds(sid * RPT, RPT)],
                        psum_out.at[cid, pl.ds(sid * RPT, RPT), pl.ds(0, d)])

    return sc_kernel, NC, NW, NCH, NNCH, NQ


def _tc_score(psum, q, pself, qneg3, bu, bi, bineg2, B, d, n_negs, NC):
    BLK = 2048
    grid = (B // BLK,)

    def body(psum_ref, q_ref, pself_ref, qneg_ref, bu_ref, bi_ref, bineg_ref,
             r_ref, rneg_ref):
        p_sum = psum_ref[0, :, :d]
        for c in range(1, NC):
            p_sum = p_sum + psum_ref[c, :, :d]
        p_ctx = p_sum - pself_ref[:, :d]
        pq = jnp.sum(p_ctx * q_ref[:, :d], axis=1)
        r_ref[...] = bu_ref[...] + bi_ref[...] + pq
        pqn = jnp.sum(p_ctx[:, None, :] * qneg_ref[:, :, :d], axis=2)
        rneg_ref[...] = bu_ref[...][:, None] + bineg_ref[...] + pqn

    return pl.pallas_call(
        body,
        grid=grid,
        in_specs=[
            pl.BlockSpec((NC, BLK, 128), lambda i: (0, i, 0)),
            pl.BlockSpec((BLK, 128), lambda i: (i, 0)),
            pl.BlockSpec((BLK, 128), lambda i: (i, 0)),
            pl.BlockSpec((BLK, n_negs, 128), lambda i: (i, 0, 0)),
            pl.BlockSpec((BLK,), lambda i: (i,)),
            pl.BlockSpec((BLK,), lambda i: (i,)),
            pl.BlockSpec((BLK, n_negs), lambda i: (i, 0)),
        ],
        out_specs=[
            pl.BlockSpec((BLK,), lambda i: (i,)),
            pl.BlockSpec((BLK, n_negs), lambda i: (i, 0)),
        ],
        out_shape=[
            jax.ShapeDtypeStruct((B,), jnp.float32),
            jax.ShapeDtypeStruct((B, n_negs), jnp.float32),
        ],
    )(psum, q, pself, qneg3, bu, bi, bineg2)


def kernel(P_table, Q_table, b_u, b_i, I, U, I_neg, I_U, U_idx):
    B = I.shape[0]
    n_negs = I_neg.shape[1]
    E = I_U.shape[0]
    n_items, d = P_table.shape
    n_users = b_u.shape[0]

    sc_kernel, NC, NW, NCH, NNCH, NQ = _make_sc_gather(
        n_items, n_users, B, E, d, n_negs)

    iu1 = I_U.astype(jnp.int32)
    us1 = U_idx.astype(jnp.int32)
    i1 = I.astype(jnp.int32)
    u1 = U.astype(jnp.int32)
    ineg1 = I_neg.astype(jnp.int32).reshape(-1)
    zeros = jnp.zeros((_K, d), jnp.float32)

    psum, q, pself, qneg, bu, bi, bineg = sc_kernel(
        P_table, Q_table, b_u, b_i, iu1, us1, i1, u1, ineg1, zeros)

    qneg3 = qneg.reshape(B, n_negs, 128)
    bineg2 = bineg.reshape(B, n_negs)
    r, rneg = _tc_score(psum, q, pself, qneg3, bu, bi, bineg2, B, d, n_negs, NC)
    return (r, rneg)
